# Initial kernel scaffold; baseline (speedup 1.0000x reference)
import jax
import jax.numpy as jnp
from jax.experimental import pallas as pl


def kernel(coords, sample_coords, data, sample_data, W0, b0, g0, beta0, W1, b1, g1, beta1):
    raise NotImplementedError("write your pallas kernel here")


# speedup vs baseline: 17.1804x; 17.1804x over previous
import functools

import jax
import jax.numpy as jnp
from jax.experimental import pallas as pl


_HIGH = jax.lax.Precision.HIGHEST


def _k1_body(coords_ref, sct_ref, data_ref, sd_ref, w0_ref, b0_ref,
             y1_ref, ssum_ref, ssq_ref):
    c = coords_ref[0]
    sct = sct_ref[0]
    R = c.shape[0]
    S = sct.shape[1]

    ab = jax.lax.dot_general(c, sct, (((1,), (0,)), ((), ())),
                             preferred_element_type=jnp.float32)
    cn = jnp.sum(c * c, axis=1, keepdims=True)
    sn = jnp.sum(sct * sct, axis=0, keepdims=True)
    sq = (-2.0 * ab + cn) + sn

    iota = jax.lax.broadcasted_iota(jnp.int32, (R, S), 1)
    d = sq
    wsum = jnp.zeros((R, S), jnp.float32)
    norm = jnp.zeros((R, 1), jnp.float32)
    for _ in range(3):
        m = jnp.min(d, axis=1, keepdims=True)
        first = jnp.min(jnp.where(d == m, iota, S), axis=1,
                        keepdims=True)
        sel = iota == first
        r = 1.0 / (m + 1e-8)
        wsum = jnp.where(sel, r, wsum)
        norm = norm + r
        d = jnp.where(sel, jnp.inf, d)
    wmat = wsum / norm

    interp = jax.lax.dot_general(wmat, sd_ref[0], (((1,), (0,)), ((), ())),
                                 preferred_element_type=jnp.float32,
                                 precision=_HIGH)
    D1 = data_ref.shape[2]
    y = (jax.lax.dot_general(data_ref[0], w0_ref[:D1], (((1,), (0,)), ((), ())),
                             preferred_element_type=jnp.float32)
         + jax.lax.dot_general(interp, w0_ref[D1:], (((1,), (0,)), ((), ())),
                               preferred_element_type=jnp.float32)
         + b0_ref[:])
    y1_ref[0] = y

    bs = jnp.sum(y, axis=0, keepdims=True)
    bq = jnp.sum(y * y, axis=0, keepdims=True)
    first_step = (pl.program_id(0) == 0) & (pl.program_id(1) == 0)

    @pl.when(first_step)
    def _():
        ssum_ref[:] = bs
        ssq_ref[:] = bq

    @pl.when(jnp.logical_not(first_step))
    def _():
        ssum_ref[:] += bs
        ssq_ref[:] += bq


def _k2_body(y1_ref, sc_ref, sh_ref, w1_ref, b1_ref,
             y2_ref, ssum_ref, ssq_ref):
    x = jax.nn.relu(y1_ref[0] * sc_ref[:] + sh_ref[:])
    y = jax.lax.dot_general(x, w1_ref[:], (((1,), (0,)), ((), ())),
                            preferred_element_type=jnp.float32) + b1_ref[:]
    y2_ref[0] = y

    bs = jnp.sum(y, axis=0, keepdims=True)
    bq = jnp.sum(y * y, axis=0, keepdims=True)
    first_step = (pl.program_id(0) == 0) & (pl.program_id(1) == 0)

    @pl.when(first_step)
    def _():
        ssum_ref[:] = bs
        ssq_ref[:] = bq

    @pl.when(jnp.logical_not(first_step))
    def _():
        ssum_ref[:] += bs
        ssq_ref[:] += bq


def _k3_body(y2_ref, sc_ref, sh_ref, out_ref):
    out_ref[0] = jax.nn.relu(y2_ref[0] * sc_ref[:] + sh_ref[:])


def _scale_shift(ssum, ssq, g, beta, count):
    mean = ssum[0] / count
    var = ssq[0] / count - mean * mean
    a = g / jnp.sqrt(var + 1e-5)
    c = beta - mean * a
    return a.reshape(1, -1), c.reshape(1, -1)


@functools.partial(jax.jit, static_argnames=("row_block",))
def _forward_impl(coords, sample_coords, data, sample_data,
                  W0, b0, g0, beta0, W1, b1, g1, beta1, row_block=512):
    B, N, _ = coords.shape
    S = sample_coords.shape[1]
    D1 = data.shape[2]
    D2 = sample_data.shape[2]
    H0 = W0.shape[1]
    H1 = W1.shape[1]
    R = row_block
    grid = (B, N // R)

    sct = sample_coords.transpose(0, 2, 1)

    y1, ssum1, ssq1 = pl.pallas_call(
        _k1_body,
        grid=grid,
        in_specs=[
            pl.BlockSpec((1, R, 3), lambda b, i: (b, i, 0)),
            pl.BlockSpec((1, 3, S), lambda b, i: (b, 0, 0)),
            pl.BlockSpec((1, R, D1), lambda b, i: (b, i, 0)),
            pl.BlockSpec((1, S, D2), lambda b, i: (b, 0, 0)),
            pl.BlockSpec((D1 + D2, H0), lambda b, i: (0, 0)),
            pl.BlockSpec((1, H0), lambda b, i: (0, 0)),
        ],
        out_specs=[
            pl.BlockSpec((1, R, H0), lambda b, i: (b, i, 0)),
            pl.BlockSpec((1, H0), lambda b, i: (0, 0)),
            pl.BlockSpec((1, H0), lambda b, i: (0, 0)),
        ],
        out_shape=[
            jax.ShapeDtypeStruct((B, N, H0), jnp.float32),
            jax.ShapeDtypeStruct((1, H0), jnp.float32),
            jax.ShapeDtypeStruct((1, H0), jnp.float32),
        ],
    )(coords, sct, data, sample_data, W0, b0.reshape(1, -1))

    a1, c1 = _scale_shift(ssum1, ssq1, g0, beta0, float(B * N))

    y2, ssum2, ssq2 = pl.pallas_call(
        _k2_body,
        grid=grid,
        in_specs=[
            pl.BlockSpec((1, R, H0), lambda b, i: (b, i, 0)),
            pl.BlockSpec((1, H0), lambda b, i: (0, 0)),
            pl.BlockSpec((1, H0), lambda b, i: (0, 0)),
            pl.BlockSpec((H0, H1), lambda b, i: (0, 0)),
            pl.BlockSpec((1, H1), lambda b, i: (0, 0)),
        ],
        out_specs=[
            pl.BlockSpec((1, R, H1), lambda b, i: (b, i, 0)),
            pl.BlockSpec((1, H1), lambda b, i: (0, 0)),
            pl.BlockSpec((1, H1), lambda b, i: (0, 0)),
        ],
        out_shape=[
            jax.ShapeDtypeStruct((B, N, H1), jnp.float32),
            jax.ShapeDtypeStruct((1, H1), jnp.float32),
            jax.ShapeDtypeStruct((1, H1), jnp.float32),
        ],
    )(y1, a1, c1, W1, b1.reshape(1, -1))

    a2, c2 = _scale_shift(ssum2, ssq2, g1, beta1, float(B * N))

    out = pl.pallas_call(
        _k3_body,
        grid=grid,
        in_specs=[
            pl.BlockSpec((1, R, H1), lambda b, i: (b, i, 0)),
            pl.BlockSpec((1, H1), lambda b, i: (0, 0)),
            pl.BlockSpec((1, H1), lambda b, i: (0, 0)),
        ],
        out_specs=pl.BlockSpec((1, R, H1), lambda b, i: (b, i, 0)),
        out_shape=jax.ShapeDtypeStruct((B, N, H1), jnp.float32),
    )(y2, a2, c2)

    return out


def kernel(coords, sample_coords, data, sample_data,
           W0, b0, g0, beta0, W1, b1, g1, beta1):
    return _forward_impl(coords, sample_coords, data, sample_data,
                         W0, b0, g0, beta0, W1, b1, g1, beta1)
